# baseline (speedup 1.0000x reference)
import jax
import jax.numpy as jnp
from jax.experimental import pallas as pl
from jax.experimental.pallas import tpu as pltpu

_TB = 256


def _gather_kernel(ids_ref, mask_ref, vocab_ref, out_ref):
    tb = out_ref.shape[0]
    for mi in range(tb):
        idx = ids_ref[mi, 0]
        m = mask_ref[mi, 0]
        out_ref[mi, 0] = vocab_ref[idx, 0] * m


def kernel(tokens, vocab, mask):
    assert tokens.ndim == 2
    V, D = vocab.shape
    d0, d1 = tokens.shape
    N = d0 * d1

    tb = _TB if N >= _TB else max(8, pl.cdiv(N, 8) * 8)
    n_pad = pl.cdiv(N, tb) * tb
    pad = n_pad - N

    ids = tokens.reshape(-1).astype(jnp.int32)
    m = mask.reshape(-1).astype(jnp.float32)
    if pad:
        ids = jnp.pad(ids, (0, pad))
        m = jnp.pad(m, (0, pad))
    ids = ids.reshape(n_pad, 1)
    m = m.reshape(n_pad, 1)
    vocab3 = vocab.reshape(V, 1, D)

    grid = n_pad // tb
    table_bytes = V * D * jnp.dtype(vocab.dtype).itemsize
    tile_bytes = tb * D * 4
    vmem_limit = int(min(64 * 1024 * 1024,
                         2 * table_bytes + 4 * tile_bytes + (4 << 20)))

    out = pl.pallas_call(
        _gather_kernel,
        out_shape=jax.ShapeDtypeStruct((n_pad, 1, D), vocab.dtype),
        grid=(grid,),
        in_specs=[
            pl.BlockSpec((tb, 1), lambda i: (i, 0), memory_space=pltpu.SMEM),
            pl.BlockSpec((tb, 1), lambda i: (i, 0), memory_space=pltpu.SMEM),
            pl.BlockSpec((V, 1, D), lambda i: (0, 0, 0)),
        ],
        out_specs=pl.BlockSpec((tb, 1, D), lambda i: (i, 0, 0)),
        compiler_params=pltpu.CompilerParams(
            dimension_semantics=("parallel",),
            vmem_limit_bytes=vmem_limit,
        ),
    )(ids, m, vocab3)

    return out[:N].reshape(d0, d1, D)
